# baseline (device time: 11887 ns/iter reference)
import jax
import jax.numpy as jnp
from jax import lax
from jax.experimental import pallas as pl
from jax.experimental.pallas import tpu as pltpu

Z = 4


def kernel(x):
    m, n = x.shape
    b = n // Z

    def body(x_ref, out_ref, stage_ref, send_sems, recv_sems, copy_sem):
        my_x = lax.axis_index("x")
        my_y = lax.axis_index("y")
        my_z = lax.axis_index("z")

        stage_ref[...] = x_ref[...].astype(jnp.bfloat16)

        own = pltpu.make_async_copy(
            stage_ref.at[:, pl.ds(my_z * b, b)],
            out_ref.at[pl.ds(my_z * m, m), :],
            copy_sem,
        )
        own.start()

        barrier_sem = pltpu.get_barrier_semaphore()
        for d in range(1, Z):
            pl.semaphore_signal(
                barrier_sem, inc=1,
                device_id=(my_x, my_y, lax.rem(my_z + d, Z)),
                device_id_type=pl.DeviceIdType.MESH,
            )
        pl.semaphore_wait(barrier_sem, Z - 1)

        for z in range(Z):

            @pl.when(my_z == z)
            def _(z=z):
                dests = sorted((p for p in range(Z) if p != z),
                               key=lambda p: -abs(p - z))
                sends = []
                for p in dests:
                    d = (p - z) % Z
                    rdma = pltpu.make_async_remote_copy(
                        src_ref=stage_ref.at[:, p * b:(p + 1) * b],
                        dst_ref=out_ref.at[z * m:(z + 1) * m, :],
                        send_sem=send_sems.at[d - 1],
                        recv_sem=recv_sems.at[d - 1],
                        device_id=(my_x, my_y, p),
                        device_id_type=pl.DeviceIdType.MESH,
                    )
                    rdma.start()
                    sends.append(rdma)

                for d in range(1, Z):
                    s = (z - d) % Z
                    recv = pltpu.make_async_remote_copy(
                        src_ref=out_ref.at[s * m:(s + 1) * m, :],
                        dst_ref=out_ref.at[s * m:(s + 1) * m, :],
                        send_sem=send_sems.at[d - 1],
                        recv_sem=recv_sems.at[d - 1],
                        device_id=(my_x, my_y, s),
                        device_id_type=pl.DeviceIdType.MESH,
                    )
                    recv.wait_recv()

                for rdma in sends:
                    rdma.wait_send()

        own.wait()

    out_shape = jax.ShapeDtypeStruct((Z * m, b), jnp.bfloat16)
    return pl.pallas_call(
        body,
        out_shape=out_shape,
        in_specs=[pl.BlockSpec(memory_space=pltpu.VMEM)],
        out_specs=pl.BlockSpec(memory_space=pltpu.HBM),
        scratch_shapes=[
            pltpu.VMEM((m, n), jnp.bfloat16),
            pltpu.SemaphoreType.DMA((Z - 1,)),
            pltpu.SemaphoreType.DMA((Z - 1,)),
            pltpu.SemaphoreType.DMA,
        ],
        compiler_params=pltpu.CompilerParams(collective_id=0),
    )(x)


# device time: 11883 ns/iter; 1.0003x vs baseline; 1.0003x over previous
import jax
import jax.numpy as jnp
from jax import lax
from jax.experimental import pallas as pl
from jax.experimental.pallas import tpu as pltpu

Z = 4


def kernel(x):
    m, n = x.shape
    b = n // Z

    def body(x_ref, out_ref, xv_ref, stage_ref, send_sems, recv_sems,
             copy_sem, in_sem):
        my_x = lax.axis_index("x")
        my_y = lax.axis_index("y")
        my_z = lax.axis_index("z")

        in_dma = pltpu.make_async_copy(x_ref, xv_ref, in_sem)
        in_dma.start()

        barrier_sem = pltpu.get_barrier_semaphore()
        for d in range(1, Z):
            pl.semaphore_signal(
                barrier_sem, inc=1,
                device_id=(my_x, my_y, lax.rem(my_z + d, Z)),
                device_id_type=pl.DeviceIdType.MESH,
            )

        in_dma.wait()
        stage_ref[...] = xv_ref[...].astype(jnp.bfloat16)

        own = pltpu.make_async_copy(
            stage_ref.at[:, pl.ds(my_z * b, b)],
            out_ref.at[pl.ds(my_z * m, m), :],
            copy_sem,
        )
        own.start()

        pl.semaphore_wait(barrier_sem, Z - 1)

        for z in range(Z):

            @pl.when(my_z == z)
            def _(z=z):
                dests = sorted((p for p in range(Z) if p != z),
                               key=lambda p: -abs(p - z))
                sends = []
                for p in dests:
                    d = (p - z) % Z
                    rdma = pltpu.make_async_remote_copy(
                        src_ref=stage_ref.at[:, p * b:(p + 1) * b],
                        dst_ref=out_ref.at[z * m:(z + 1) * m, :],
                        send_sem=send_sems.at[d - 1],
                        recv_sem=recv_sems.at[d - 1],
                        device_id=(my_x, my_y, p),
                        device_id_type=pl.DeviceIdType.MESH,
                    )
                    rdma.start()
                    sends.append(rdma)

                for d in range(1, Z):
                    s = (z - d) % Z
                    recv = pltpu.make_async_remote_copy(
                        src_ref=out_ref.at[s * m:(s + 1) * m, :],
                        dst_ref=out_ref.at[s * m:(s + 1) * m, :],
                        send_sem=send_sems.at[d - 1],
                        recv_sem=recv_sems.at[d - 1],
                        device_id=(my_x, my_y, s),
                        device_id_type=pl.DeviceIdType.MESH,
                    )
                    recv.wait_recv()

                for rdma in sends:
                    rdma.wait_send()

        own.wait()

    out_shape = jax.ShapeDtypeStruct((Z * m, b), jnp.bfloat16)
    return pl.pallas_call(
        body,
        out_shape=out_shape,
        in_specs=[pl.BlockSpec(memory_space=pltpu.HBM)],
        out_specs=pl.BlockSpec(memory_space=pltpu.HBM),
        scratch_shapes=[
            pltpu.VMEM((m, n), x.dtype),
            pltpu.VMEM((m, n), jnp.bfloat16),
            pltpu.SemaphoreType.DMA((Z - 1,)),
            pltpu.SemaphoreType.DMA((Z - 1,)),
            pltpu.SemaphoreType.DMA,
            pltpu.SemaphoreType.DMA,
        ],
        compiler_params=pltpu.CompilerParams(collective_id=0),
    )(x)


# device time: 11853 ns/iter; 1.0029x vs baseline; 1.0025x over previous
import jax
import jax.numpy as jnp
from jax import lax
from jax.experimental import pallas as pl
from jax.experimental.pallas import tpu as pltpu

Z = 4


def kernel(x):
    m, n = x.shape
    b = n // Z

    def body(x_ref, out_ref, stage_ref, send_sems, recv_sems):
        my_x = lax.axis_index("x")
        my_y = lax.axis_index("y")
        my_z = lax.axis_index("z")

        stage_ref[...] = x_ref[...].astype(jnp.bfloat16)

        barrier_sem = pltpu.get_barrier_semaphore()
        for d in range(1, Z):
            pl.semaphore_signal(
                barrier_sem, inc=1,
                device_id=(my_x, my_y, lax.rem(my_z + d, Z)),
                device_id_type=pl.DeviceIdType.MESH,
            )
        pl.semaphore_wait(barrier_sem, Z - 1)

        for z in range(Z):

            @pl.when(my_z == z)
            def _(z=z):
                dests = sorted((p for p in range(Z) if p != z),
                               key=lambda p: -abs(p - z))
                sends = []
                for p in dests:
                    d = (p - z) % Z
                    rdma = pltpu.make_async_remote_copy(
                        src_ref=stage_ref.at[:, p * b:(p + 1) * b],
                        dst_ref=out_ref.at[z * m:(z + 1) * m, :],
                        send_sem=send_sems.at[d - 1],
                        recv_sem=recv_sems.at[d - 1],
                        device_id=(my_x, my_y, p),
                        device_id_type=pl.DeviceIdType.MESH,
                    )
                    rdma.start()
                    sends.append(rdma)

                out_ref[z * m:(z + 1) * m, :] = stage_ref[:, z * b:(z + 1) * b]

                for d in range(1, Z):
                    s = (z - d) % Z
                    recv = pltpu.make_async_remote_copy(
                        src_ref=out_ref.at[s * m:(s + 1) * m, :],
                        dst_ref=out_ref.at[s * m:(s + 1) * m, :],
                        send_sem=send_sems.at[d - 1],
                        recv_sem=recv_sems.at[d - 1],
                        device_id=(my_x, my_y, s),
                        device_id_type=pl.DeviceIdType.MESH,
                    )
                    recv.wait_recv()

                for rdma in sends:
                    rdma.wait_send()

    out_shape = jax.ShapeDtypeStruct((Z * m, b), jnp.bfloat16)
    return pl.pallas_call(
        body,
        out_shape=out_shape,
        in_specs=[pl.BlockSpec(memory_space=pltpu.VMEM)],
        out_specs=pl.BlockSpec(memory_space=pltpu.VMEM),
        scratch_shapes=[
            pltpu.VMEM((m, n), jnp.bfloat16),
            pltpu.SemaphoreType.DMA((Z - 1,)),
            pltpu.SemaphoreType.DMA((Z - 1,)),
        ],
        compiler_params=pltpu.CompilerParams(collective_id=0),
    )(x)
